# baseline (speedup 1.0000x reference)
import functools

import jax
import jax.numpy as jnp
from jax import lax
from jax.experimental import pallas as pl
from jax.experimental.pallas import tpu as pltpu
from jax.experimental.pallas import tpu_sc as plsc

N_ROWS = 16384
N_CODES = 8192
DIM = 256
N_SPLITS = 4
SPLIT_ROWS = N_ROWS // N_SPLITS
BLOCK_ROWS = 512
N_BLOCKS = SPLIT_ROWS // BLOCK_ROWS

CHUNK = 256
N_CHUNKS = N_CODES // CHUNK


def _prep_kernel(emb_ref, e2_ref, embt_ref):
    emb = emb_ref[...]
    e2_ref[0, :] = jnp.sum(emb * emb, axis=1)
    embt_ref[...] = emb.T.astype(jnp.bfloat16)


def _prep(emb_weight):
    return pl.pallas_call(
        _prep_kernel,
        out_shape=[
            jax.ShapeDtypeStruct((1, N_CODES), jnp.float32),
            jax.ShapeDtypeStruct((DIM, N_CODES), jnp.bfloat16),
        ],
    )(emb_weight)


def _distance_argmin_kernel(x_ref, e2_ref, embt_ref, idx_ref, dsum_ref,
                            acc_ref):
    i = pl.program_id(0)

    @pl.when(i == 0)
    def _init():
        acc_ref[0, 0] = 0.0

    x_blk = x_ref[...]
    x2 = jnp.sum(x_blk * x_blk, axis=1, keepdims=True)
    xs = (-2.0 * x_blk).astype(jnp.bfloat16)

    lane = lax.broadcasted_iota(jnp.int32, (BLOCK_ROWS, 128), 1)
    val = jnp.full((BLOCK_ROWS, 128), jnp.inf, jnp.float32)
    cidx = jnp.zeros((BLOCK_ROWS, 128), jnp.int32)
    for j in range(N_CHUNKS):
        m = jnp.dot(xs, embt_ref[:, j * CHUNK:(j + 1) * CHUNK],
                    preferred_element_type=jnp.float32)
        e2c = e2_ref[0, j * CHUNK:(j + 1) * CHUNK]
        s = (x2 + e2c[None, :]) + m
        s0 = s[:, :128]
        s1 = s[:, 128:]
        c0 = lane + j * CHUNK
        h1 = s1 < s0
        sc = jnp.where(h1, s1, s0)
        cc = jnp.where(h1, c0 + 128, c0)
        better = sc < val
        cidx = jnp.where(better, cc, cidx)
        val = jnp.where(better, sc, val)

    minv = jnp.min(val, axis=1)
    sel = jnp.where(val == minv[:, None], cidx, jnp.int32(N_CODES))
    idx_ref[...] = jnp.min(sel, axis=1)
    acc_ref[0, 0] += jnp.sum(minv)

    @pl.when(i == pl.num_programs(0) - 1)
    def _fin():
        dsum_ref[0, 0] = acc_ref[0, 0]


def _nearest_codes(x_split, e2, embt):
    return pl.pallas_call(
        _distance_argmin_kernel,
        grid=(N_BLOCKS,),
        in_specs=[
            pl.BlockSpec((BLOCK_ROWS, DIM), lambda i: (i, 0)),
            pl.BlockSpec((1, N_CODES), lambda i: (0, 0)),
            pl.BlockSpec((DIM, N_CODES), lambda i: (0, 0)),
        ],
        out_specs=[
            pl.BlockSpec((BLOCK_ROWS,), lambda i: (i,)),
            pl.BlockSpec(memory_space=pltpu.SMEM),
        ],
        out_shape=[
            jax.ShapeDtypeStruct((SPLIT_ROWS,), jnp.int32),
            jax.ShapeDtypeStruct((1, 1), jnp.float32),
        ],
        scratch_shapes=[
            pltpu.SMEM((1, 1), jnp.float32),
        ],
    )(x_split, e2, embt)


_SC_CHUNK = 128


def _sc_gather_kernel(emb_hbm, idx_hbm, out_hbm,
                      idx_v, rows_a, rows_b, gsem, osem):
    n_cores = 2
    wid = lax.axis_index("s") * n_cores + lax.axis_index("c")
    rows_per_w = SPLIT_ROWS // 32
    n_chunks = rows_per_w // _SC_CHUNK
    base = wid * rows_per_w
    pltpu.sync_copy(idx_hbm.at[pl.ds(base, rows_per_w)], idx_v)
    bufs = (rows_a, rows_b)
    copies = []
    for c in range(n_chunks):
        copies.append(pltpu.async_copy(
            emb_hbm.at[idx_v.at[pl.ds(c * _SC_CHUNK, _SC_CHUNK)]],
            bufs[c % 2], gsem))
        if c > 0:
            copies[c - 1].wait()
            pltpu.async_copy(
                bufs[(c - 1) % 2],
                out_hbm.at[pl.ds(base + (c - 1) * _SC_CHUNK, _SC_CHUNK)],
                osem).wait()
    copies[n_chunks - 1].wait()
    pltpu.async_copy(
        bufs[(n_chunks - 1) % 2],
        out_hbm.at[pl.ds(base + (n_chunks - 1) * _SC_CHUNK, _SC_CHUNK)],
        osem).wait()


def _gather_rows(emb_weight, idx):
    mesh = plsc.VectorSubcoreMesh(core_axis_name="c", subcore_axis_name="s")
    k = functools.partial(
        pl.kernel,
        out_type=jax.ShapeDtypeStruct((SPLIT_ROWS, DIM), jnp.float32),
        mesh=mesh,
        scratch_types=[
            pltpu.VMEM((SPLIT_ROWS // 32,), jnp.int32),
            pltpu.VMEM((_SC_CHUNK, DIM), jnp.float32),
            pltpu.VMEM((_SC_CHUNK, DIM), jnp.float32),
            pltpu.SemaphoreType.DMA,
            pltpu.SemaphoreType.DMA,
        ],
    )(_sc_gather_kernel)
    return k(emb_weight, idx)


def kernel(x, emb_weight):
    e2, embt = _prep(emb_weight)
    parts = []
    dsums = []
    for q in range(N_SPLITS):
        xq = lax.slice_in_dim(x, q * SPLIT_ROWS, (q + 1) * SPLIT_ROWS, axis=0)
        idx_q, dsum_q = _nearest_codes(xq, e2, embt)
        parts.append(_gather_rows(emb_weight, idx_q))
        dsums.append(dsum_q[0, 0])
    quantized = jnp.concatenate(parts, axis=0)
    dsum = dsums[0] + dsums[1] + dsums[2] + dsums[3]
    loss = dsum * (1.25 / (N_ROWS * DIM))
    return (quantized, loss)


# speedup vs baseline: 1.0332x; 1.0332x over previous
import functools

import jax
import jax.numpy as jnp
from jax import lax
from jax.experimental import pallas as pl
from jax.experimental.pallas import tpu as pltpu
from jax.experimental.pallas import tpu_sc as plsc

N_ROWS = 16384
N_CODES = 8192
DIM = 256
N_SPLITS = 1
SPLIT_ROWS = N_ROWS // N_SPLITS
BLOCK_ROWS = 512
N_BLOCKS = SPLIT_ROWS // BLOCK_ROWS

CHUNK = 256
N_CHUNKS = N_CODES // CHUNK
GROUP_ROWS = 256


def _prep_kernel(emb_ref, e2_ref, embt_ref):
    emb = emb_ref[...]
    e2_ref[0, :] = jnp.sum(emb * emb, axis=1)
    embt_ref[...] = emb.T.astype(jnp.bfloat16)


def _prep(emb_weight):
    return pl.pallas_call(
        _prep_kernel,
        out_shape=[
            jax.ShapeDtypeStruct((1, N_CODES), jnp.float32),
            jax.ShapeDtypeStruct((DIM, N_CODES), jnp.bfloat16),
        ],
    )(emb_weight)


def _distance_argmin_kernel(x_ref, e2_ref, embt_ref, idx_ref, dsum_ref,
                            acc_ref):
    i = pl.program_id(0)

    @pl.when(i == 0)
    def _init():
        acc_ref[0, 0] = 0.0

    x_blk = x_ref[...]
    x2 = jnp.sum(x_blk * x_blk, axis=1, keepdims=True)
    xs = (-2.0 * x_blk).astype(jnp.bfloat16)

    lane = lax.broadcasted_iota(jnp.int32, (GROUP_ROWS, 128), 1)
    blk_sum = jnp.float32(0.0)
    for g in range(BLOCK_ROWS // GROUP_ROWS):
        xg = xs[g * GROUP_ROWS:(g + 1) * GROUP_ROWS]
        x2g = x2[g * GROUP_ROWS:(g + 1) * GROUP_ROWS]
        val = jnp.full((GROUP_ROWS, 128), jnp.inf, jnp.float32)
        cidx = jnp.zeros((GROUP_ROWS, 128), jnp.int32)
        for j in range(N_CHUNKS):
            m = jnp.dot(xg, embt_ref[:, j * CHUNK:(j + 1) * CHUNK],
                        preferred_element_type=jnp.float32)
            e2c = e2_ref[0, j * CHUNK:(j + 1) * CHUNK]
            s = (x2g + e2c[None, :]) + m
            s0 = s[:, :128]
            s1 = s[:, 128:]
            c0 = lane + j * CHUNK
            h1 = s1 < s0
            sc = jnp.where(h1, s1, s0)
            cc = jnp.where(h1, c0 + 128, c0)
            better = sc < val
            cidx = jnp.where(better, cc, cidx)
            val = jnp.where(better, sc, val)

        minv = jnp.min(val, axis=1)
        sel = jnp.where(val == minv[:, None], cidx, jnp.int32(N_CODES))
        idx_ref[g * GROUP_ROWS:(g + 1) * GROUP_ROWS] = jnp.min(sel, axis=1)
        blk_sum = blk_sum + jnp.sum(minv)
    acc_ref[0, 0] += blk_sum

    @pl.when(i == pl.num_programs(0) - 1)
    def _fin():
        dsum_ref[0, 0] = acc_ref[0, 0]


def _nearest_codes(x_split, e2, embt):
    return pl.pallas_call(
        _distance_argmin_kernel,
        grid=(N_BLOCKS,),
        in_specs=[
            pl.BlockSpec((BLOCK_ROWS, DIM), lambda i: (i, 0)),
            pl.BlockSpec((1, N_CODES), lambda i: (0, 0)),
            pl.BlockSpec((DIM, N_CODES), lambda i: (0, 0)),
        ],
        out_specs=[
            pl.BlockSpec((BLOCK_ROWS,), lambda i: (i,)),
            pl.BlockSpec(memory_space=pltpu.SMEM),
        ],
        out_shape=[
            jax.ShapeDtypeStruct((SPLIT_ROWS,), jnp.int32),
            jax.ShapeDtypeStruct((1, 1), jnp.float32),
        ],
        scratch_shapes=[
            pltpu.SMEM((1, 1), jnp.float32),
        ],
    )(x_split, e2, embt)


_SC_CHUNK = 128


def _sc_gather_kernel(emb_hbm, idx_hbm, out_hbm,
                      idx_v, rows_a, rows_b, gsem, osem):
    n_cores = 2
    wid = lax.axis_index("s") * n_cores + lax.axis_index("c")
    rows_per_w = SPLIT_ROWS // 32
    n_chunks = rows_per_w // _SC_CHUNK
    base = wid * rows_per_w
    pltpu.sync_copy(idx_hbm.at[pl.ds(base, rows_per_w)], idx_v)
    bufs = (rows_a, rows_b)
    copies = []
    for c in range(n_chunks):
        copies.append(pltpu.async_copy(
            emb_hbm.at[idx_v.at[pl.ds(c * _SC_CHUNK, _SC_CHUNK)]],
            bufs[c % 2], gsem))
        if c > 0:
            copies[c - 1].wait()
            pltpu.async_copy(
                bufs[(c - 1) % 2],
                out_hbm.at[pl.ds(base + (c - 1) * _SC_CHUNK, _SC_CHUNK)],
                osem).wait()
    copies[n_chunks - 1].wait()
    pltpu.async_copy(
        bufs[(n_chunks - 1) % 2],
        out_hbm.at[pl.ds(base + (n_chunks - 1) * _SC_CHUNK, _SC_CHUNK)],
        osem).wait()


def _gather_rows(emb_weight, idx):
    mesh = plsc.VectorSubcoreMesh(core_axis_name="c", subcore_axis_name="s")
    k = functools.partial(
        pl.kernel,
        out_type=jax.ShapeDtypeStruct((SPLIT_ROWS, DIM), jnp.float32),
        mesh=mesh,
        scratch_types=[
            pltpu.VMEM((SPLIT_ROWS // 32,), jnp.int32),
            pltpu.VMEM((_SC_CHUNK, DIM), jnp.float32),
            pltpu.VMEM((_SC_CHUNK, DIM), jnp.float32),
            pltpu.SemaphoreType.DMA,
            pltpu.SemaphoreType.DMA,
        ],
    )(_sc_gather_kernel)
    return k(emb_weight, idx)


def kernel(x, emb_weight):
    e2, embt = _prep(emb_weight)
    parts = []
    dsums = []
    for q in range(N_SPLITS):
        xq = lax.slice_in_dim(x, q * SPLIT_ROWS, (q + 1) * SPLIT_ROWS, axis=0)
        idx_q, dsum_q = _nearest_codes(xq, e2, embt)
        parts.append(_gather_rows(emb_weight, idx_q))
        dsums.append(dsum_q[0, 0])
    quantized = parts[0] if len(parts) == 1 else jnp.concatenate(parts, axis=0)
    dsum = dsums[0]
    for d in dsums[1:]:
        dsum = dsum + d
    loss = dsum * (1.25 / (N_ROWS * DIM))
    return (quantized, loss)


# speedup vs baseline: 1.0523x; 1.0185x over previous
import functools

import jax
import jax.numpy as jnp
from jax import lax
from jax.experimental import pallas as pl
from jax.experimental.pallas import tpu as pltpu
from jax.experimental.pallas import tpu_sc as plsc

N_ROWS = 16384
N_CODES = 8192
DIM = 256
N_SPLITS = 1
SPLIT_ROWS = N_ROWS // N_SPLITS
BLOCK_ROWS = 512
N_BLOCKS = SPLIT_ROWS // BLOCK_ROWS

CHUNK = 256
N_CHUNKS = N_CODES // CHUNK
GROUP_ROWS = 256


def _prep_kernel(emb_ref, e2_ref, embt_ref):
    emb = emb_ref[...]
    e2_ref[0, :] = jnp.sum(emb * emb, axis=1)
    embt_ref[...] = emb.T.astype(jnp.bfloat16)


def _prep(emb_weight):
    return pl.pallas_call(
        _prep_kernel,
        out_shape=[
            jax.ShapeDtypeStruct((1, N_CODES), jnp.float32),
            jax.ShapeDtypeStruct((DIM, N_CODES), jnp.bfloat16),
        ],
    )(emb_weight)


def _distance_argmin_kernel(x_ref, e2_ref, embt_ref, idx_ref, dsum_ref,
                            acc_ref):
    i = pl.program_id(0)

    @pl.when(i == 0)
    def _init():
        acc_ref[0, 0] = 0.0

    x_blk = x_ref[...]
    x2 = jnp.sum(x_blk * x_blk, axis=1, keepdims=True)
    xs = (-2.0 * x_blk).astype(jnp.bfloat16)

    lane = lax.broadcasted_iota(jnp.int32, (GROUP_ROWS, 128), 1)
    blk_sum = jnp.float32(0.0)
    for g in range(BLOCK_ROWS // GROUP_ROWS):
        xg = xs[g * GROUP_ROWS:(g + 1) * GROUP_ROWS]
        x2g = x2[g * GROUP_ROWS:(g + 1) * GROUP_ROWS]
        val = jnp.full((GROUP_ROWS, 128), jnp.inf, jnp.float32)
        cidx = jnp.zeros((GROUP_ROWS, 128), jnp.int32)
        for j in range(N_CHUNKS):
            m = jnp.dot(xg, embt_ref[:, j * CHUNK:(j + 1) * CHUNK],
                        preferred_element_type=jnp.float32)
            e2c = e2_ref[0, j * CHUNK:(j + 1) * CHUNK]
            s = (x2g + e2c[None, :]) + m
            s0 = s[:, :128]
            s1 = s[:, 128:]
            c0 = lane + j * CHUNK
            h1 = s1 < s0
            sc = jnp.where(h1, s1, s0)
            cc = jnp.where(h1, c0 + 128, c0)
            better = sc < val
            cidx = jnp.where(better, cc, cidx)
            val = jnp.where(better, sc, val)

        minv = jnp.min(val, axis=1)
        sel = jnp.where(val == minv[:, None], cidx, jnp.int32(N_CODES))
        idx_ref[g * GROUP_ROWS:(g + 1) * GROUP_ROWS, :] = jnp.min(
            sel, axis=1, keepdims=True)
        blk_sum = blk_sum + jnp.sum(minv)
    acc_ref[0, 0] += blk_sum

    @pl.when(i == pl.num_programs(0) - 1)
    def _fin():
        dsum_ref[0, 0] = acc_ref[0, 0]


def _nearest_codes(x_split, e2, embt):
    return pl.pallas_call(
        _distance_argmin_kernel,
        grid=(N_BLOCKS,),
        in_specs=[
            pl.BlockSpec((BLOCK_ROWS, DIM), lambda i: (i, 0)),
            pl.BlockSpec((1, N_CODES), lambda i: (0, 0)),
            pl.BlockSpec((DIM, N_CODES), lambda i: (0, 0)),
        ],
        out_specs=[
            pl.BlockSpec((BLOCK_ROWS, 1), lambda i: (i, 0)),
            pl.BlockSpec(memory_space=pltpu.SMEM),
        ],
        out_shape=[
            jax.ShapeDtypeStruct((SPLIT_ROWS, 1), jnp.int32),
            jax.ShapeDtypeStruct((1, 1), jnp.float32),
        ],
        scratch_shapes=[
            pltpu.SMEM((1, 1), jnp.float32),
        ],
    )(x_split, e2, embt)


_SC_CHUNK = 128


def _sc_gather_kernel(emb_hbm, idx_hbm, out_hbm,
                      idx_v, rows_a, rows_b, gsem, osem):
    n_cores = 2
    wid = lax.axis_index("s") * n_cores + lax.axis_index("c")
    rows_per_w = SPLIT_ROWS // 32
    n_chunks = rows_per_w // _SC_CHUNK
    base = wid * rows_per_w
    pltpu.sync_copy(idx_hbm.at[pl.ds(base, rows_per_w)], idx_v)
    bufs = (rows_a, rows_b)
    copies = []
    for c in range(n_chunks):
        copies.append(pltpu.async_copy(
            emb_hbm.at[idx_v.at[pl.ds(c * _SC_CHUNK, _SC_CHUNK)]],
            bufs[c % 2], gsem))
        if c > 0:
            copies[c - 1].wait()
            pltpu.async_copy(
                bufs[(c - 1) % 2],
                out_hbm.at[pl.ds(base + (c - 1) * _SC_CHUNK, _SC_CHUNK)],
                osem).wait()
    copies[n_chunks - 1].wait()
    pltpu.async_copy(
        bufs[(n_chunks - 1) % 2],
        out_hbm.at[pl.ds(base + (n_chunks - 1) * _SC_CHUNK, _SC_CHUNK)],
        osem).wait()


def _gather_rows(emb_weight, idx):
    mesh = plsc.VectorSubcoreMesh(core_axis_name="c", subcore_axis_name="s")
    k = functools.partial(
        pl.kernel,
        out_type=jax.ShapeDtypeStruct((SPLIT_ROWS, DIM), jnp.float32),
        mesh=mesh,
        scratch_types=[
            pltpu.VMEM((SPLIT_ROWS // 32,), jnp.int32),
            pltpu.VMEM((_SC_CHUNK, DIM), jnp.float32),
            pltpu.VMEM((_SC_CHUNK, DIM), jnp.float32),
            pltpu.SemaphoreType.DMA,
            pltpu.SemaphoreType.DMA,
        ],
    )(_sc_gather_kernel)
    return k(emb_weight, idx)


def kernel(x, emb_weight):
    e2, embt = _prep(emb_weight)
    parts = []
    dsums = []
    for q in range(N_SPLITS):
        xq = lax.slice_in_dim(x, q * SPLIT_ROWS, (q + 1) * SPLIT_ROWS, axis=0)
        idx_q, dsum_q = _nearest_codes(xq, e2, embt)
        idx_q = idx_q.reshape((SPLIT_ROWS,))
        parts.append(_gather_rows(emb_weight, idx_q))
        dsums.append(dsum_q[0, 0])
    quantized = parts[0] if len(parts) == 1 else jnp.concatenate(parts, axis=0)
    dsum = dsums[0]
    for d in dsums[1:]:
        dsum = dsum + d
    loss = dsum * (1.25 / (N_ROWS * DIM))
    return (quantized, loss)


# speedup vs baseline: 1.0534x; 1.0010x over previous
import functools

import jax
import jax.numpy as jnp
from jax import lax
from jax.experimental import pallas as pl
from jax.experimental.pallas import tpu as pltpu
from jax.experimental.pallas import tpu_sc as plsc

N_ROWS = 16384
N_CODES = 8192
DIM = 256
N_SPLITS = 1
SPLIT_ROWS = N_ROWS // N_SPLITS
BLOCK_ROWS = 1024
N_BLOCKS = SPLIT_ROWS // BLOCK_ROWS

CHUNK = 256
N_CHUNKS = N_CODES // CHUNK
GROUP_ROWS = 256


def _prep_kernel(emb_ref, e2_ref, embt_ref):
    emb = emb_ref[...]
    e2_ref[0, :] = jnp.sum(emb * emb, axis=1)
    embt_ref[...] = emb.T.astype(jnp.bfloat16)


def _prep(emb_weight):
    return pl.pallas_call(
        _prep_kernel,
        out_shape=[
            jax.ShapeDtypeStruct((1, N_CODES), jnp.float32),
            jax.ShapeDtypeStruct((DIM, N_CODES), jnp.bfloat16),
        ],
    )(emb_weight)


def _distance_argmin_kernel(x_ref, e2_ref, embt_ref, idx_ref, dsum_ref,
                            acc_ref):
    i = pl.program_id(0)

    @pl.when(i == 0)
    def _init():
        acc_ref[0, 0] = 0.0

    x_blk = x_ref[...]
    x2 = jnp.sum(x_blk * x_blk, axis=1, keepdims=True)
    xs = (-2.0 * x_blk).astype(jnp.bfloat16)

    lane = lax.broadcasted_iota(jnp.int32, (GROUP_ROWS, 128), 1)
    blk_sum = jnp.float32(0.0)
    for g in range(BLOCK_ROWS // GROUP_ROWS):
        xg = xs[g * GROUP_ROWS:(g + 1) * GROUP_ROWS]
        x2g = x2[g * GROUP_ROWS:(g + 1) * GROUP_ROWS]
        val = jnp.full((GROUP_ROWS, 128), jnp.inf, jnp.float32)
        cidx = jnp.zeros((GROUP_ROWS, 128), jnp.int32)
        for j in range(N_CHUNKS):
            m = jnp.dot(xg, embt_ref[:, j * CHUNK:(j + 1) * CHUNK],
                        preferred_element_type=jnp.float32)
            e2c = e2_ref[0, j * CHUNK:(j + 1) * CHUNK]
            s = (x2g + e2c[None, :]) + m
            s0 = s[:, :128]
            s1 = s[:, 128:]
            c0 = lane + j * CHUNK
            h1 = s1 < s0
            sc = jnp.where(h1, s1, s0)
            cc = jnp.where(h1, c0 + 128, c0)
            better = sc < val
            cidx = jnp.where(better, cc, cidx)
            val = jnp.where(better, sc, val)

        minv = jnp.min(val, axis=1)
        sel = jnp.where(val == minv[:, None], cidx, jnp.int32(N_CODES))
        idx_ref[g * GROUP_ROWS:(g + 1) * GROUP_ROWS, :] = jnp.min(
            sel, axis=1, keepdims=True)
        blk_sum = blk_sum + jnp.sum(minv)
    acc_ref[0, 0] += blk_sum

    @pl.when(i == pl.num_programs(0) - 1)
    def _fin():
        dsum_ref[0, 0] = acc_ref[0, 0]


def _nearest_codes(x_split, e2, embt):
    return pl.pallas_call(
        _distance_argmin_kernel,
        grid=(N_BLOCKS,),
        in_specs=[
            pl.BlockSpec((BLOCK_ROWS, DIM), lambda i: (i, 0)),
            pl.BlockSpec((1, N_CODES), lambda i: (0, 0)),
            pl.BlockSpec((DIM, N_CODES), lambda i: (0, 0)),
        ],
        out_specs=[
            pl.BlockSpec((BLOCK_ROWS, 1), lambda i: (i, 0)),
            pl.BlockSpec(memory_space=pltpu.SMEM),
        ],
        out_shape=[
            jax.ShapeDtypeStruct((SPLIT_ROWS, 1), jnp.int32),
            jax.ShapeDtypeStruct((1, 1), jnp.float32),
        ],
        scratch_shapes=[
            pltpu.SMEM((1, 1), jnp.float32),
        ],
    )(x_split, e2, embt)


_SC_CHUNK = 128


def _sc_gather_kernel(emb_hbm, idx_hbm, out_hbm,
                      idx_v, rows_a, rows_b, rows_c, gsem,
                      osem_a, osem_b, osem_c):
    n_cores = 2
    wid = lax.axis_index("s") * n_cores + lax.axis_index("c")
    rows_per_w = SPLIT_ROWS // 32
    n_chunks = rows_per_w // _SC_CHUNK
    base = wid * rows_per_w
    pltpu.sync_copy(idx_hbm.at[pl.ds(base, rows_per_w)], idx_v)
    bufs = (rows_a, rows_b, rows_c)
    osems = (osem_a, osem_b, osem_c)
    gathers = []
    writes = []
    for c in range(n_chunks):
        if c >= 3:
            writes[c - 3].wait()
        gathers.append(pltpu.async_copy(
            emb_hbm.at[idx_v.at[pl.ds(c * _SC_CHUNK, _SC_CHUNK)]],
            bufs[c % 3], gsem))
        if c > 0:
            gathers[c - 1].wait()
            writes.append(pltpu.async_copy(
                bufs[(c - 1) % 3],
                out_hbm.at[pl.ds(base + (c - 1) * _SC_CHUNK, _SC_CHUNK)],
                osems[(c - 1) % 3]))
    gathers[n_chunks - 1].wait()
    writes.append(pltpu.async_copy(
        bufs[(n_chunks - 1) % 3],
        out_hbm.at[pl.ds(base + (n_chunks - 1) * _SC_CHUNK, _SC_CHUNK)],
        osems[(n_chunks - 1) % 3]))
    for c in range(max(0, n_chunks - 3), n_chunks):
        writes[c].wait()


def _gather_rows(emb_weight, idx):
    mesh = plsc.VectorSubcoreMesh(core_axis_name="c", subcore_axis_name="s")
    k = functools.partial(
        pl.kernel,
        out_type=jax.ShapeDtypeStruct((SPLIT_ROWS, DIM), jnp.float32),
        mesh=mesh,
        scratch_types=[
            pltpu.VMEM((SPLIT_ROWS // 32,), jnp.int32),
            pltpu.VMEM((_SC_CHUNK, DIM), jnp.float32),
            pltpu.VMEM((_SC_CHUNK, DIM), jnp.float32),
            pltpu.VMEM((_SC_CHUNK, DIM), jnp.float32),
            pltpu.SemaphoreType.DMA,
            pltpu.SemaphoreType.DMA,
            pltpu.SemaphoreType.DMA,
            pltpu.SemaphoreType.DMA,
        ],
    )(_sc_gather_kernel)
    return k(emb_weight, idx)


def kernel(x, emb_weight):
    e2, embt = _prep(emb_weight)
    parts = []
    dsums = []
    for q in range(N_SPLITS):
        xq = lax.slice_in_dim(x, q * SPLIT_ROWS, (q + 1) * SPLIT_ROWS, axis=0)
        idx_q, dsum_q = _nearest_codes(xq, e2, embt)
        idx_q = idx_q.reshape((SPLIT_ROWS,))
        parts.append(_gather_rows(emb_weight, idx_q))
        dsums.append(dsum_q[0, 0])
    quantized = parts[0] if len(parts) == 1 else jnp.concatenate(parts, axis=0)
    dsum = dsums[0]
    for d in dsums[1:]:
        dsum = dsum + d
    loss = dsum * (1.25 / (N_ROWS * DIM))
    return (quantized, loss)
